# Initial kernel scaffold; baseline (speedup 1.0000x reference)
import jax
import jax.numpy as jnp
from jax.experimental import pallas as pl


def kernel(x, ema_vals):
    raise NotImplementedError("write your pallas kernel here")


# speedup vs baseline: 34.4555x; 34.4555x over previous
import functools

import jax
import jax.numpy as jnp
import numpy as np
from jax import lax
from jax.experimental import pallas as pl
from jax.experimental.pallas import tpu as pltpu
from jax.experimental.pallas import tpu_sc as plsc

N_TOTAL = 4096 * 2048
NUM_WORKERS = 32
PER_WORKER = N_TOTAL // NUM_WORKERS
CHUNK = 16384
CHUNKS_PER_WORKER = PER_WORKER // CHUNK
HIST1_BINS = 65536
HIST2_BINS = 32768
VREGS_PER_CHUNK = CHUNK // 16
UNROLL = 8

_ALPHA = np.float32(0.01)
_ONE_MINUS_ALPHA = np.float32(1.0) - np.float32(0.01)

_r = np.array([0.05, 0.95], dtype=np.float32)
_idx = np.clip(np.round(_r * np.float32(N_TOTAL - 1)), 0, N_TOTAL - 1)
K_LO = int(_idx[0])
K_HI = int(_idx[1])


def _sc_hist1_body(x_hbm, hist_hbm, hist_v, buf0, buf1, sem0, sem1):
    wid = lax.axis_index("s") * 2 + lax.axis_index("c")
    zeros16 = jnp.zeros((16,), jnp.int32)
    ones16 = jnp.ones((16,), jnp.int32)

    def zero_body(z, carry):
        off = z * 256
        for t in range(16):
            hist_v[pl.ds(off + t * 16, 16)] = zeros16
        return carry

    lax.fori_loop(0, HIST1_BINS // 256, zero_body, 0)

    bufs = (buf0, buf1)
    sems = (sem0, sem1)
    pltpu.async_copy(x_hbm.at[wid, 0], bufs[0], sems[0])
    for c in range(CHUNKS_PER_WORKER):
        if c + 1 < CHUNKS_PER_WORKER:
            pltpu.async_copy(x_hbm.at[wid, c + 1], bufs[(c + 1) % 2], sems[(c + 1) % 2])
        buf = bufs[c % 2]
        pltpu.make_async_copy(x_hbm.at[wid, c], buf, sems[c % 2]).wait()

        def chunk_body(j, carry):
            s = j * (16 * UNROLL)
            for t in range(UNROLL):
                v = buf[pl.ds(s + t * 16, 16)]
                b = lax.bitcast_convert_type(v, jnp.int32)
                bucket = (b >> 16) + jnp.int32(32768)
                plsc.addupdate_scatter(hist_v, [bucket], ones16)
            return carry

        lax.fori_loop(0, VREGS_PER_CHUNK // UNROLL, chunk_body, 0)

    pltpu.sync_copy(hist_v, hist_hbm.at[wid])


def _sc_hist2_body(x_hbm, pfx_hbm, hist_hbm, hist_v, pfx_v, buf0, buf1, sem0, sem1):
    wid = lax.axis_index("s") * 2 + lax.axis_index("c")
    zeros16 = jnp.zeros((16,), jnp.int32)
    ones16 = jnp.ones((16,), jnp.int32)

    pltpu.sync_copy(pfx_hbm, pfx_v)
    p0 = pfx_v[0, :]
    p1 = pfx_v[1, :]

    def zero_body(z, carry):
        off = z * 256
        for t in range(16):
            hist_v[pl.ds(off + t * 16, 16)] = zeros16
        return carry

    lax.fori_loop(0, (2 * HIST2_BINS) // 256, zero_body, 0)

    bufs = (buf0, buf1)
    sems = (sem0, sem1)
    pltpu.async_copy(x_hbm.at[wid, 0], bufs[0], sems[0])
    for c in range(CHUNKS_PER_WORKER):
        if c + 1 < CHUNKS_PER_WORKER:
            pltpu.async_copy(x_hbm.at[wid, c + 1], bufs[(c + 1) % 2], sems[(c + 1) % 2])
        buf = bufs[c % 2]
        pltpu.make_async_copy(x_hbm.at[wid, c], buf, sems[c % 2]).wait()

        def chunk_body(j, carry):
            s = j * (16 * UNROLL)
            for t in range(UNROLL):
                v = buf[pl.ds(s + t * 16, 16)]
                b = lax.bitcast_convert_type(v, jnp.int32)
                bucket = (b >> 16) + jnp.int32(32768)
                bin2 = jnp.bitwise_and(b >> 1, jnp.int32(0x7FFF))
                plsc.addupdate_scatter(hist_v, [bin2], ones16, mask=bucket == p0)
                plsc.addupdate_scatter(
                    hist_v, [bin2 + jnp.int32(HIST2_BINS)], ones16, mask=bucket == p1
                )
            return carry

        lax.fori_loop(0, VREGS_PER_CHUNK // UNROLL, chunk_body, 0)

    pltpu.sync_copy(hist_v, hist_hbm.at[wid])


def _search(h2d, idx2d, k, hi0, steps):
    def step(_, lohi):
        lo, hi = lohi
        mid = (lo + hi) // 2
        cnt = jnp.sum(jnp.where(idx2d < mid, h2d, 0))
        take = cnt <= k
        return jnp.where(take, mid, lo), jnp.where(take, hi, mid)

    lo, _ = lax.fori_loop(
        0, steps, step, (jnp.int32(0), jnp.int32(hi0))
    )
    cnt_below = jnp.sum(jnp.where(idx2d < lo, h2d, 0))
    return lo, cnt_below


def _bin_idx(rows):
    r = lax.broadcasted_iota(jnp.int32, (rows, 128), 0)
    l = lax.broadcasted_iota(jnp.int32, (rows, 128), 1)
    return r * 128 + l


def _ord1_idx():
    raw = _bin_idx(512)
    return jnp.where(raw < 32768, jnp.int32(32767) - raw, raw)


def _tca_body(h_ref, out_ref):
    hs = jnp.sum(h_ref[...], axis=0)
    oidx = _ord1_idx()
    bo_lo, _ = _search(hs, oidx, jnp.int32(K_LO), HIST1_BINS, 17)
    bo_hi, _ = _search(hs, oidx, jnp.int32(K_HI), HIST1_BINS, 17)
    b_lo = jnp.where(bo_lo < 32768, jnp.int32(32767) - bo_lo, bo_lo)
    b_hi = jnp.where(bo_hi < 32768, jnp.int32(32767) - bo_hi, bo_hi)
    rows = lax.broadcasted_iota(jnp.int32, (8, 128), 0)
    out_ref[...] = jnp.where(rows == 0, b_lo, jnp.where(rows == 1, b_hi, 0))


def _tcb_body(h1_ref, h2_ref, ema_ref, off_ref, scale_ref):
    hs1 = jnp.sum(h1_ref[...], axis=0)
    oidx1 = _ord1_idx()
    bo_lo, below_lo = _search(hs1, oidx1, jnp.int32(K_LO), HIST1_BINS, 17)
    bo_hi, below_hi = _search(hs1, oidx1, jnp.int32(K_HI), HIST1_BINS, 17)
    b_lo = jnp.where(bo_lo < 32768, jnp.int32(32767) - bo_lo, bo_lo)
    b_hi = jnp.where(bo_hi < 32768, jnp.int32(32767) - bo_hi, bo_hi)
    r_lo = jnp.int32(K_LO) - below_lo
    r_hi = jnp.int32(K_HI) - below_hi

    hs2 = jnp.sum(h2_ref[...], axis=0)
    idx2 = _bin_idx(256)
    oidx2_lo = jnp.where(b_lo < 32768, jnp.int32(32767) - idx2, idx2)
    oidx2_hi = jnp.where(b_hi < 32768, jnp.int32(32767) - idx2, idx2)
    so_lo, _ = _search(hs2[0:256], oidx2_lo, r_lo, HIST2_BINS, 16)
    so_hi, _ = _search(hs2[256:512], oidx2_hi, r_hi, HIST2_BINS, 16)
    s_lo = jnp.where(b_lo < 32768, jnp.int32(32767) - so_lo, so_lo)
    s_hi = jnp.where(b_hi < 32768, jnp.int32(32767) - so_hi, so_hi)

    bits_lo = jnp.bitwise_or((b_lo - jnp.int32(32768)) << 16, s_lo << 1)
    bits_hi = jnp.bitwise_or((b_hi - jnp.int32(32768)) << 16, s_hi << 1)
    q = lax.bitcast_convert_type(
        jnp.stack(
            [
                jnp.broadcast_to(bits_lo, (1, 1)),
                jnp.broadcast_to(bits_hi, (1, 1)),
            ]
        ),
        jnp.float32,
    )
    e0 = ema_ref[0, 0]
    e1 = ema_ref[0, 1]
    ne0 = _ALPHA * q[0] + _ONE_MINUS_ALPHA * e0
    ne1 = _ALPHA * q[1] + _ONE_MINUS_ALPHA * e1
    off_ref[...] = ne0
    scale_ref[...] = jnp.maximum(ne1 - ne0, jnp.float32(1.0))


@functools.lru_cache(maxsize=1)
def _sc_kernels():
    mesh = plsc.VectorSubcoreMesh(
        core_axis_name="c", subcore_axis_name="s", num_cores=2, num_subcores=16
    )
    params = pltpu.CompilerParams(needs_layout_passes=False)
    sc1 = pl.kernel(
        _sc_hist1_body,
        out_type=jax.ShapeDtypeStruct((NUM_WORKERS, HIST1_BINS), jnp.int32),
        mesh=mesh,
        compiler_params=params,
        scratch_types=[
            pltpu.VMEM((HIST1_BINS,), jnp.int32),
            pltpu.VMEM((CHUNK,), jnp.float32),
            pltpu.VMEM((CHUNK,), jnp.float32),
            pltpu.SemaphoreType.DMA,
            pltpu.SemaphoreType.DMA,
        ],
    )
    sc2 = pl.kernel(
        _sc_hist2_body,
        out_type=jax.ShapeDtypeStruct((NUM_WORKERS, 2 * HIST2_BINS), jnp.int32),
        mesh=mesh,
        compiler_params=params,
        scratch_types=[
            pltpu.VMEM((2 * HIST2_BINS,), jnp.int32),
            pltpu.VMEM((2, 16), jnp.int32),
            pltpu.VMEM((CHUNK,), jnp.float32),
            pltpu.VMEM((CHUNK,), jnp.float32),
            pltpu.SemaphoreType.DMA,
            pltpu.SemaphoreType.DMA,
        ],
    )
    return sc1, sc2


_tca = pl.pallas_call(
    _tca_body,
    out_shape=jax.ShapeDtypeStruct((8, 128), jnp.int32),
)

_tcb = pl.pallas_call(
    _tcb_body,
    out_shape=[
        jax.ShapeDtypeStruct((1, 1), jnp.float32),
        jax.ShapeDtypeStruct((1, 1), jnp.float32),
    ],
)


def kernel(x, ema_vals):
    sc_hist1, sc_hist2 = _sc_kernels()
    x3 = x.astype(jnp.float32).reshape(NUM_WORKERS, CHUNKS_PER_WORKER, CHUNK)
    hist1 = sc_hist1(x3)
    h1 = hist1.reshape(NUM_WORKERS, 512, 128)
    meta = _tca(h1)
    prefix = meta[0:2, 0:16]
    hist2 = sc_hist2(x3, prefix)
    h2 = hist2.reshape(NUM_WORKERS, 512, 128)
    off, scale = _tcb(h1, h2, ema_vals.astype(jnp.float32).reshape(1, 2))
    return (off.reshape(()), scale.reshape(()))


# speedup vs baseline: 82.8720x; 2.4052x over previous
import functools

import jax
import jax.numpy as jnp
import numpy as np
from jax import lax
from jax.experimental import pallas as pl
from jax.experimental.pallas import tpu as pltpu
from jax.experimental.pallas import tpu_sc as plsc

N_TOTAL = 4096 * 2048
NUM_WORKERS = 32
PER_WORKER = N_TOTAL // NUM_WORKERS
CHUNK = 16384
CHUNKS_PER_WORKER = PER_WORKER // CHUNK
HIST1_BINS = 16384
HIST2_BINS = 16384
VREGS_PER_CHUNK = CHUNK // 16
UNROLL = 8

_ALPHA = np.float32(0.01)
_ONE_MINUS_ALPHA = np.float32(1.0) - np.float32(0.01)

_r = np.array([0.05, 0.95], dtype=np.float32)
_idx = np.clip(np.round(_r * np.float32(N_TOTAL - 1)), 0, N_TOTAL - 1)
K_LO = int(_idx[0])
K_HI = int(_idx[1])


def _zero_hists(refs, words_each):
    zeros16 = jnp.zeros((16,), jnp.int32)
    n_refs = len(refs)

    def zero_body(z, carry):
        off = z * 128
        for r in refs:
            for t in range(8):
                r[pl.ds(off + t * 16, 16)] = zeros16
        return carry

    lax.fori_loop(0, words_each // 128, zero_body, 0)
    del n_refs


def _sc_hist1_body(x_hbm, hist_hbm, h0, h1, h2, h3, buf0, buf1, sem0, sem1):
    wid = lax.axis_index("s") * 2 + lax.axis_index("c")
    ones16 = jnp.ones((16,), jnp.int32)
    hists = (h0, h1, h2, h3)

    _zero_hists(hists, HIST1_BINS)

    bufs = (buf0, buf1)
    sems = (sem0, sem1)
    pltpu.async_copy(x_hbm.at[wid, 0], bufs[0], sems[0])
    for c in range(CHUNKS_PER_WORKER):
        if c + 1 < CHUNKS_PER_WORKER:
            pltpu.async_copy(x_hbm.at[wid, c + 1], bufs[(c + 1) % 2], sems[(c + 1) % 2])
        buf = bufs[c % 2]
        pltpu.make_async_copy(x_hbm.at[wid, c], buf, sems[c % 2]).wait()

        @plsc.parallel_loop(0, CHUNK, 16 * UNROLL)
        def chunk_body(s):
            for t in range(UNROLL):
                v = buf[pl.ds(s + t * 16, 16)]
                b = lax.bitcast_convert_type(v, jnp.int32)
                bucket = (b >> 18) + jnp.int32(8192)
                plsc.addupdate_scatter(hists[t % 4], [bucket], ones16)

    def sum_body(z, carry):
        off = z * 128
        for t in range(8):
            d = pl.ds(off + t * 16, 16)
            h0[d] = h0[d] + h1[d] + h2[d] + h3[d]
        return carry

    lax.fori_loop(0, HIST1_BINS // 128, sum_body, 0)
    pltpu.sync_copy(h0, hist_hbm.at[wid])


def _sc_hist2_body(x_hbm, pfx_hbm, hist_hbm, hA, hB, pfx_v, buf0, buf1, sem0, sem1):
    wid = lax.axis_index("s") * 2 + lax.axis_index("c")
    ones16 = jnp.ones((16,), jnp.int32)

    pltpu.sync_copy(pfx_hbm, pfx_v)
    p0 = pfx_v[0, :]
    p1 = pfx_v[1, :]

    _zero_hists((hA, hB), 2 * HIST2_BINS)

    bufs = (buf0, buf1)
    sems = (sem0, sem1)
    pltpu.async_copy(x_hbm.at[wid, 0], bufs[0], sems[0])
    for c in range(CHUNKS_PER_WORKER):
        if c + 1 < CHUNKS_PER_WORKER:
            pltpu.async_copy(x_hbm.at[wid, c + 1], bufs[(c + 1) % 2], sems[(c + 1) % 2])
        buf = bufs[c % 2]
        pltpu.make_async_copy(x_hbm.at[wid, c], buf, sems[c % 2]).wait()

        @plsc.parallel_loop(0, CHUNK, 16 * UNROLL)
        def chunk_body(s):
            for t in range(UNROLL):
                v = buf[pl.ds(s + t * 16, 16)]
                b = lax.bitcast_convert_type(v, jnp.int32)
                bucket = (b >> 18) + jnp.int32(8192)
                bin2 = jnp.bitwise_and(b >> 4, jnp.int32(0x3FFF))
                ra = (hA, hB)[t % 2]
                rb = (hB, hA)[t % 2]
                plsc.addupdate_scatter(ra, [bin2], ones16, mask=bucket == p0)
                plsc.addupdate_scatter(
                    rb, [bin2 + jnp.int32(HIST2_BINS)], ones16, mask=bucket == p1
                )

    def sum_body(z, carry):
        off = z * 128
        for t in range(8):
            d = pl.ds(off + t * 16, 16)
            hA[d] = hA[d] + hB[d]
        return carry

    lax.fori_loop(0, (2 * HIST2_BINS) // 128, sum_body, 0)
    pltpu.sync_copy(hA, hist_hbm.at[wid])


@functools.lru_cache(maxsize=1)
def _sc_kernels():
    mesh = plsc.VectorSubcoreMesh(
        core_axis_name="c", subcore_axis_name="s", num_cores=2, num_subcores=16
    )
    params = pltpu.CompilerParams(needs_layout_passes=False)
    sc1 = pl.kernel(
        _sc_hist1_body,
        out_type=jax.ShapeDtypeStruct((NUM_WORKERS, HIST1_BINS), jnp.int32),
        mesh=mesh,
        compiler_params=params,
        scratch_types=[
            pltpu.VMEM((HIST1_BINS,), jnp.int32),
            pltpu.VMEM((HIST1_BINS,), jnp.int32),
            pltpu.VMEM((HIST1_BINS,), jnp.int32),
            pltpu.VMEM((HIST1_BINS,), jnp.int32),
            pltpu.VMEM((CHUNK,), jnp.float32),
            pltpu.VMEM((CHUNK,), jnp.float32),
            pltpu.SemaphoreType.DMA,
            pltpu.SemaphoreType.DMA,
        ],
    )
    sc2 = pl.kernel(
        _sc_hist2_body,
        out_type=jax.ShapeDtypeStruct((NUM_WORKERS, 2 * HIST2_BINS), jnp.int32),
        mesh=mesh,
        compiler_params=params,
        scratch_types=[
            pltpu.VMEM((2 * HIST2_BINS,), jnp.int32),
            pltpu.VMEM((2 * HIST2_BINS,), jnp.int32),
            pltpu.VMEM((2, 16), jnp.int32),
            pltpu.VMEM((CHUNK,), jnp.float32),
            pltpu.VMEM((CHUNK,), jnp.float32),
            pltpu.SemaphoreType.DMA,
            pltpu.SemaphoreType.DMA,
        ],
    )
    return sc1, sc2


def _search(h2d, idx2d, k, hi0, steps):
    def step(_, lohi):
        lo, hi = lohi
        mid = (lo + hi) // 2
        cnt = jnp.sum(jnp.where(idx2d < mid, h2d, 0))
        take = cnt <= k
        return jnp.where(take, mid, lo), jnp.where(take, hi, mid)

    lo, _ = lax.fori_loop(0, steps, step, (jnp.int32(0), jnp.int32(hi0)))
    cnt_below = jnp.sum(jnp.where(idx2d < lo, h2d, 0))
    return lo, cnt_below


def _bin_idx(rows):
    r = lax.broadcasted_iota(jnp.int32, (rows, 128), 0)
    l = lax.broadcasted_iota(jnp.int32, (rows, 128), 1)
    return r * 128 + l


def _ord1_idx():
    raw = _bin_idx(HIST1_BINS // 128)
    return jnp.where(raw < 8192, jnp.int32(8191) - raw, raw)


def _tca_body(h_ref, out_ref):
    hs = jnp.sum(h_ref[...], axis=0)
    oidx = _ord1_idx()
    bo_lo, _ = _search(hs, oidx, jnp.int32(K_LO), HIST1_BINS, 15)
    bo_hi, _ = _search(hs, oidx, jnp.int32(K_HI), HIST1_BINS, 15)
    b_lo = jnp.where(bo_lo < 8192, jnp.int32(8191) - bo_lo, bo_lo)
    b_hi = jnp.where(bo_hi < 8192, jnp.int32(8191) - bo_hi, bo_hi)
    rows = lax.broadcasted_iota(jnp.int32, (8, 128), 0)
    out_ref[...] = jnp.where(rows == 0, b_lo, jnp.where(rows == 1, b_hi, 0))


def _tcb_body(h1_ref, h2_ref, ema_ref, off_ref, scale_ref):
    hs1 = jnp.sum(h1_ref[...], axis=0)
    oidx1 = _ord1_idx()
    bo_lo, below_lo = _search(hs1, oidx1, jnp.int32(K_LO), HIST1_BINS, 15)
    bo_hi, below_hi = _search(hs1, oidx1, jnp.int32(K_HI), HIST1_BINS, 15)
    b_lo = jnp.where(bo_lo < 8192, jnp.int32(8191) - bo_lo, bo_lo)
    b_hi = jnp.where(bo_hi < 8192, jnp.int32(8191) - bo_hi, bo_hi)
    r_lo = jnp.int32(K_LO) - below_lo
    r_hi = jnp.int32(K_HI) - below_hi

    hs2 = jnp.sum(h2_ref[...], axis=0)
    idx2 = _bin_idx(HIST2_BINS // 128)
    oidx2_lo = jnp.where(b_lo < 8192, jnp.int32(HIST2_BINS - 1) - idx2, idx2)
    oidx2_hi = jnp.where(b_hi < 8192, jnp.int32(HIST2_BINS - 1) - idx2, idx2)
    so_lo, _ = _search(hs2[0:128], oidx2_lo, r_lo, HIST2_BINS, 15)
    so_hi, _ = _search(hs2[128:256], oidx2_hi, r_hi, HIST2_BINS, 15)
    s_lo = jnp.where(b_lo < 8192, jnp.int32(HIST2_BINS - 1) - so_lo, so_lo)
    s_hi = jnp.where(b_hi < 8192, jnp.int32(HIST2_BINS - 1) - so_hi, so_hi)

    bits_lo = jnp.bitwise_or((b_lo - jnp.int32(8192)) << 18, s_lo << 4)
    bits_hi = jnp.bitwise_or((b_hi - jnp.int32(8192)) << 18, s_hi << 4)
    q = lax.bitcast_convert_type(
        jnp.stack(
            [
                jnp.broadcast_to(bits_lo, (1, 1)),
                jnp.broadcast_to(bits_hi, (1, 1)),
            ]
        ),
        jnp.float32,
    )
    e0 = ema_ref[0, 0]
    e1 = ema_ref[0, 1]
    ne0 = _ALPHA * q[0] + _ONE_MINUS_ALPHA * e0
    ne1 = _ALPHA * q[1] + _ONE_MINUS_ALPHA * e1
    off_ref[...] = ne0
    scale_ref[...] = jnp.maximum(ne1 - ne0, jnp.float32(1.0))


_tca = pl.pallas_call(
    _tca_body,
    out_shape=jax.ShapeDtypeStruct((8, 128), jnp.int32),
)

_tcb = pl.pallas_call(
    _tcb_body,
    out_shape=[
        jax.ShapeDtypeStruct((1, 1), jnp.float32),
        jax.ShapeDtypeStruct((1, 1), jnp.float32),
    ],
)


def kernel(x, ema_vals):
    sc_hist1, sc_hist2 = _sc_kernels()
    x3 = x.astype(jnp.float32).reshape(NUM_WORKERS, CHUNKS_PER_WORKER, CHUNK)
    hist1 = sc_hist1(x3)
    h1 = hist1.reshape(NUM_WORKERS, HIST1_BINS // 128, 128)
    meta = _tca(h1)
    prefix = meta[0:2, 0:16]
    hist2 = sc_hist2(x3, prefix)
    h2 = hist2.reshape(NUM_WORKERS, (2 * HIST2_BINS) // 128, 128)
    off, scale = _tcb(h1, h2, ema_vals.astype(jnp.float32).reshape(1, 2))
    return (off.reshape(()), scale.reshape(()))
